# baseline (speedup 1.0000x reference)
import functools

import jax
import jax.numpy as jnp
from jax import lax
from jax.experimental import pallas as pl
from jax.experimental.pallas import tpu as pltpu
from jax.experimental.pallas import tpu_sc as plsc

B = 16384
NF = 32
L = 16
NC = 2
NS = 16
NW = NC * NS
BPW = B // NW
DCH = 128
NDCH = BPW // DCH
NCH = BPW // L
NDAY = 366

_LN2 = 0.6931471805599453
_SQRT2 = 1.4142135623730951
_BETA = 0.4


def _body(user_r, item_r, tbin_r, tday_r, mc_r, mean_r, bu_r, alpha_r,
          auk_r, bcu_r, wpu_r, wpi_r, bi_r, wbitf_r, pkut_r, btday_r,
          wcu_r, gm_r, out_r,
          u_v, i_v, tb_v, td_v, mc_v, w_v,
          bu_v, al_v, me_v, bc_v, bi_v, wb_v,
          wpu_v, auk_v, wpi_v,
          pkut_t, btd_t, wcu_t, gm_v, out_v, sem):
    wid = lax.axis_index("s") * NC + lax.axis_index("c")
    base = wid * BPW

    pltpu.sync_copy(user_r.at[pl.ds(base, BPW)], u_v)
    pltpu.sync_copy(item_r.at[pl.ds(base, BPW)], i_v)
    pltpu.sync_copy(tbin_r.at[pl.ds(base, BPW)], tb_v)
    pltpu.sync_copy(tday_r.at[pl.ds(base, BPW)], td_v)
    pltpu.sync_copy(mc_r.at[pl.ds(base, BPW)], mc_v)

    def widx_body(k, c):
        sl = pl.ds(k * L, L)
        w_v[sl] = i_v[sl] * 30 + tb_v[sl]
        return c
    lax.fori_loop(0, NCH, widx_body, 0)

    copies = []
    for j in range(NDCH):
        sl = pl.ds(j * DCH, DCH)
        uix = u_v.at[sl]
        iix = i_v.at[sl]
        wix = w_v.at[sl]
        copies.append(pltpu.async_copy(bu_r.at[uix], bu_v.at[sl], sem))
        copies.append(pltpu.async_copy(alpha_r.at[uix], al_v.at[sl], sem))
        copies.append(pltpu.async_copy(mean_r.at[uix], me_v.at[sl], sem))
        copies.append(pltpu.async_copy(bcu_r.at[uix], bc_v.at[sl], sem))
        copies.append(pltpu.async_copy(wpu_r.at[uix], wpu_v.at[sl], sem))
        copies.append(pltpu.async_copy(auk_r.at[uix], auk_v.at[sl], sem))
        copies.append(pltpu.async_copy(bi_r.at[iix], bi_v.at[sl], sem))
        copies.append(pltpu.async_copy(wpi_r.at[iix], wpi_v.at[sl], sem))
        copies.append(pltpu.async_copy(wbitf_r.at[wix], wb_v.at[sl], sem))

    pltpu.sync_copy(pkut_r, pkut_t)
    pltpu.sync_copy(btday_r, btd_t)
    pltpu.sync_copy(wcu_r, wcu_t)
    pltpu.sync_copy(gm_r, gm_v)

    for c in copies:
        c.wait()

    gm16 = gm_v[...]

    def chunk_body(k, c):
        b16 = k * L
        sl = pl.ds(b16, L)
        diff = td_v[sl].astype(jnp.float32) - me_v[sl]
        sgn = jnp.sign(diff)
        t = jnp.abs(diff)
        bits = lax.bitcast_convert_type(t, jnp.int32)
        e_i = (bits >> 23) - 127
        m = lax.bitcast_convert_type((bits & 0x7FFFFF) | 0x3F800000,
                                     jnp.float32)
        big = m > _SQRT2
        m = jnp.where(big, m * 0.5, m)
        e_f = e_i.astype(jnp.float32) + jnp.where(big, 1.0, 0.0)
        z = (m - 1.0) / (m + 1.0)
        z2 = z * z
        poly = 1.0 + z2 * ((1.0 / 3.0) + z2 * ((1.0 / 5.0) + z2 * (1.0 / 7.0)))
        ln_t = e_f * _LN2 + 2.0 * z * poly
        devt = sgn * jnp.exp(_BETA * ln_t)

        mc16 = mc_v[sl]
        btd16 = plsc.load_gather(btd_t, [mc16])
        wcu16 = plsc.load_gather(wcu_t, [mc16])
        acc = (gm16 + bu_v[sl] + al_v[sl] * devt + btd16
               + (bi_v[sl] + wb_v[sl]) * (bc_v[sl] + wcu16))

        e16 = b16 + lax.iota(jnp.int32, L)
        for f in range(NF):
            f16 = jnp.full((L,), f, jnp.int32)
            uvf = plsc.load_gather(wpu_v, [e16, f16])
            akf = plsc.load_gather(auk_v, [e16, f16])
            pkf = plsc.load_gather(pkut_t, [mc16, f16])
            ivf = plsc.load_gather(wpi_v, [e16, f16])
            acc = acc + (uvf + akf * devt + pkf) * ivf
        out_v[sl] = acc
        return c

    lax.fori_loop(0, NCH, chunk_body, 0)

    pltpu.sync_copy(out_v, out_r.at[pl.ds(base, BPW)])


@jax.jit
def _run(user, item, tbin, tday, mc, mean_ud, bu, alpha, auk, bcu,
         wpu, wpi, bi, wbit_flat, pkut, btday, wcu, gm16):
    mesh = plsc.VectorSubcoreMesh(core_axis_name="c", subcore_axis_name="s")
    f = functools.partial(
        pl.kernel,
        out_type=jax.ShapeDtypeStruct((B,), jnp.float32),
        mesh=mesh,
        compiler_params=pltpu.CompilerParams(needs_layout_passes=False,
                                             use_tc_tiling_on_sc=False),
        scratch_types=[
            pltpu.VMEM((BPW,), jnp.int32),
            pltpu.VMEM((BPW,), jnp.int32),
            pltpu.VMEM((BPW,), jnp.int32),
            pltpu.VMEM((BPW,), jnp.int32),
            pltpu.VMEM((BPW,), jnp.int32),
            pltpu.VMEM((BPW,), jnp.int32),
            pltpu.VMEM((BPW,), jnp.float32),
            pltpu.VMEM((BPW,), jnp.float32),
            pltpu.VMEM((BPW,), jnp.float32),
            pltpu.VMEM((BPW,), jnp.float32),
            pltpu.VMEM((BPW,), jnp.float32),
            pltpu.VMEM((BPW,), jnp.float32),
            pltpu.VMEM((BPW, NF), jnp.float32),
            pltpu.VMEM((BPW, NF), jnp.float32),
            pltpu.VMEM((BPW, NF), jnp.float32),
            pltpu.VMEM((NDAY, NF), jnp.float32),
            pltpu.VMEM((NDAY,), jnp.float32),
            pltpu.VMEM((NDAY,), jnp.float32),
            pltpu.VMEM((L,), jnp.float32),
            pltpu.VMEM((BPW,), jnp.float32),
            pltpu.SemaphoreType.DMA,
        ],
    )(_body)
    return f(user, item, tbin, tday, mc, mean_ud, bu, alpha, auk, bcu,
             wpu, wpi, bi, wbit_flat, pkut, btday, wcu, gm16)


def kernel(user, item, tbin, tday, mean_ud, global_mean, maxday_cat,
           WPI, WPU, BU, BI, WBIT, Alpha, AlphaUK, WPUKT, BTDay, BCU, WCU):
    gm16 = jnp.broadcast_to(jnp.float32(global_mean), (L,))
    return _run(user.astype(jnp.int32), item.astype(jnp.int32),
                tbin.astype(jnp.int32), tday.astype(jnp.int32),
                maxday_cat.astype(jnp.int32), mean_ud, BU, Alpha, AlphaUK,
                BCU, WPU, WPI, BI, WBIT.reshape(-1), WPUKT, BTDay, WCU,
                gm16)
